# baseline (speedup 1.0000x reference)
import functools

import jax
import jax.numpy as jnp
from jax import lax
from jax.experimental import pallas as pl
from jax.experimental.pallas import tpu as pltpu
from jax.experimental.pallas import tpu_sc as plsc

N_UNITS = 100000
K_SUB = 5
BATCH = 1024
N_REC = 4096

NC = 2
NS = 16
NW = NC * NS
ROWS_PER_W = BATCH // NW
NIDX = N_REC * K_SUB
LANES = 16
NCHUNKS = N_REC // LANES


def _sc_body(e_hbm, idx_hbm, cb_hbm, out_hbm,
             rowbuf, idxv, cbv, or0, or1,
             semo0, semo1):
    cid = lax.axis_index("c")
    sid = lax.axis_index("s")
    wid = sid * NC + cid
    base = wid * ROWS_PER_W

    pltpu.sync_copy(idx_hbm, idxv)
    pltpu.sync_copy(cb_hbm.at[pl.ds(base * LANES, ROWS_PER_W * LANES)], cbv)

    orows = (or0, or1)
    osems = (semo0, semo1)

    def compute(row_local, orow):
        c16 = cbv[pl.ds(row_local * LANES, LANES)] * jnp.float32(K_SUB)

        def m_body(m, carry):
            o = m * LANES
            ivec = idxv[pl.ds(o, LANES)]
            acc = plsc.load_gather(rowbuf, [ivec])
            for k in range(1, K_SUB):
                ivec = idxv[pl.ds(k * N_REC + o, LANES)]
                acc = acc + plsc.load_gather(rowbuf, [ivec])
            t = c16 - acc
            p = 1.0 / (1.0 + jnp.exp(-t))
            orow[pl.ds(o, LANES)] = p
            return carry

        lax.fori_loop(0, NCHUNKS, m_body, 0)

    def i_body(i2, carry):
        for s in range(2):
            i = i2 * 2 + s
            r_abs = base + i
            pltpu.sync_copy(e_hbm.at[r_abs], rowbuf)

            @pl.when(i2 >= 1)
            def _wait_out():
                pltpu.make_async_copy(
                    orows[s], out_hbm.at[base], osems[s]).wait()

            compute(i, orows[s])
            pltpu.async_copy(orows[s], out_hbm.at[r_abs], osems[s])
        return carry

    lax.fori_loop(0, ROWS_PER_W // 2, i_body, 0)

    pltpu.make_async_copy(or0, out_hbm.at[base], semo0).wait()
    pltpu.make_async_copy(or1, out_hbm.at[base], semo1).wait()


@jax.jit
def _sc_call(energies, idx_km, cb):
    mesh = plsc.VectorSubcoreMesh(core_axis_name="c", subcore_axis_name="s")
    f = functools.partial(
        pl.kernel,
        out_type=jax.ShapeDtypeStruct((BATCH, N_REC), jnp.float32),
        mesh=mesh,
        compiler_params=pltpu.CompilerParams(needs_layout_passes=False),
        scratch_types=[
            pltpu.VMEM((N_UNITS,), jnp.float32),
            pltpu.VMEM((NIDX,), jnp.int32),
            pltpu.VMEM((ROWS_PER_W * LANES,), jnp.float32),
            pltpu.VMEM((N_REC,), jnp.float32),
            pltpu.VMEM((N_REC,), jnp.float32),
            pltpu.SemaphoreType.DMA,
            pltpu.SemaphoreType.DMA,
        ],
    )(_sc_body)
    return f(energies, idx_km, cb)


def kernel(energies, concentrations, receptor_indices):
    idx_km = receptor_indices.astype(jnp.int32).T.reshape(-1)
    cb = jnp.broadcast_to(
        concentrations.reshape(BATCH, 1), (BATCH, LANES)
    ).reshape(-1)
    return _sc_call(energies, idx_km, cb)
